# baseline (speedup 1.0000x reference)
import functools

import jax
import jax.numpy as jnp
from jax import lax
from jax.experimental import pallas as pl
from jax.experimental.pallas import tpu as pltpu
from jax.experimental.pallas import tpu_sc as plsc

B = 16384
D = 64
CH = 128
L = 16

_info = plsc.get_sparse_core_info()
NC, NS = _info.num_cores, _info.num_subcores
NW = NC * NS
BPW = B // NW
NCH = BPW // CH
NGRP = BPW // L

_mesh = plsc.VectorSubcoreMesh(core_axis_name="c", subcore_axis_name="s")


@functools.partial(
    pl.kernel,
    mesh=_mesh,
    out_type=jax.ShapeDtypeStruct((B,), jnp.float32),
    compiler_params=pltpu.CompilerParams(
        needs_layout_passes=False, use_tc_tiling_on_sc=False),
    scratch_types=[
        pltpu.VMEM((NCH, CH), jnp.int32),
        pltpu.VMEM((NCH, CH), jnp.int32),
        pltpu.VMEM((NCH, CH), jnp.int32),
        pltpu.VMEM((BPW, D), jnp.float32),
        pltpu.VMEM((BPW, D), jnp.float32),
        pltpu.VMEM((BPW, D), jnp.float32),
        pltpu.VMEM((BPW,), jnp.float32),
        pltpu.SemaphoreType.DMA,
    ],
)
def _transe_sc(head_hbm, rel_hbm, tail_hbm, ent_hbm, relt_hbm, out_hbm,
               hi_v, ri_v, ti_v, h_v, r_v, t_v, o_v, sem):
    wid = lax.axis_index("s") * NC + lax.axis_index("c")
    crow = wid * NCH

    pltpu.sync_copy(head_hbm.at[pl.ds(crow, NCH)], hi_v)
    pltpu.sync_copy(rel_hbm.at[pl.ds(crow, NCH)], ri_v)
    pltpu.sync_copy(tail_hbm.at[pl.ds(crow, NCH)], ti_v)

    copies = []
    for c in range(NCH):
        dst = pl.ds(c * CH, CH)
        copies.append(pltpu.async_copy(ent_hbm.at[hi_v.at[c]], h_v.at[dst], sem))
        copies.append(pltpu.async_copy(relt_hbm.at[ri_v.at[c]], r_v.at[dst], sem))
        copies.append(pltpu.async_copy(ent_hbm.at[ti_v.at[c]], t_v.at[dst], sem))
    for cp in copies:
        cp.wait()

    lane = lax.iota(jnp.int32, L)

    def group_body(g, carry):
        row0 = g * L
        out_vec = jnp.zeros((L,), jnp.float32)
        for i in range(L):
            row = row0 + i
            acc = jnp.zeros((L,), jnp.float32)
            for k in range(D // L):
                cs = pl.ds(k * L, L)
                h = h_v[row, cs]
                r = r_v[row, cs]
                t = t_v[row, cs]
                acc = acc + jnp.abs(h + r - t)
            out_vec = jnp.where(lane == i, jnp.sum(acc), out_vec)
        o_v[pl.ds(row0, L)] = out_vec
        return carry

    lax.fori_loop(0, NGRP, group_body, jnp.int32(0))

    pltpu.sync_copy(o_v, out_hbm.at[pl.ds(wid * BPW, BPW)])


def kernel(head, relation, tail, entity_table, relation_table):
    head2 = head.reshape(B // CH, CH)
    rel2 = relation.reshape(B // CH, CH)
    tail2 = tail.reshape(B // CH, CH)
    return _transe_sc(head2, rel2, tail2, entity_table, relation_table)
